# baseline (speedup 1.0000x reference)
import functools

import jax
import jax.numpy as jnp
from jax import lax
from jax.experimental import pallas as pl
from jax.experimental.pallas import tpu as pltpu
from jax.experimental.pallas import tpu_sc as plsc

EPS = 1e-09
N_TASKS = 1000
N_SKILLS = 64
N_SPLITS = 8
BS = 16384
D = N_SKILLS * N_SPLITS

NUM_CORES = 2
NUM_SUBCORES = 16
NUM_WORKERS = NUM_CORES * NUM_SUBCORES
B_PER_W = BS // NUM_WORKERS
CHUNK = 128
N_CHUNKS = B_PER_W // CHUNK


def _normalize_body(w_ref, out_ref):
    s = jax.nn.sigmoid(w_ref[...])
    denom = jnp.sum(s, axis=1, keepdims=True) + EPS
    out_ref[...] = s / denom


def _normalize_table(w2):
    return pl.pallas_call(
        _normalize_body,
        out_shape=jax.ShapeDtypeStruct(w2.shape, w2.dtype),
    )(w2)


_mesh = plsc.VectorSubcoreMesh(core_axis_name="c", subcore_axis_name="s")


@functools.partial(
    pl.kernel,
    mesh=_mesh,
    out_type=jax.ShapeDtypeStruct((BS, N_SPLITS, N_SKILLS), jnp.float32),
    scratch_types=[
        pltpu.VMEM((CHUNK,), jnp.int32),
        pltpu.VMEM((CHUNK, N_SPLITS, N_SKILLS), jnp.float32),
        pltpu.SemaphoreType.DMA,
    ],
    compiler_params=pltpu.CompilerParams(use_tc_tiling_on_sc=False),
)
def _sc_gather(idx_hbm, table_hbm, out_hbm, idx_v, rows_v, sem):
    wid = lax.axis_index("s") * NUM_CORES + lax.axis_index("c")
    base = wid * B_PER_W
    for c in range(N_CHUNKS):
        off = base + c * CHUNK
        pltpu.sync_copy(idx_hbm.at[pl.ds(off, CHUNK)], idx_v)
        pltpu.async_copy(table_hbm.at[idx_v], rows_v, sem).wait()
        pltpu.sync_copy(rows_v, out_hbm.at[pl.ds(off, CHUNK)])


def kernel(routing_info, weights):
    w2 = weights.reshape(N_TASKS * N_SPLITS, N_SKILLS)
    table = _normalize_table(w2).reshape(N_TASKS, N_SPLITS, N_SKILLS)
    idx = routing_info.reshape(BS).astype(jnp.int32)
    return _sc_gather(idx, table)


# speedup vs baseline: 1.2620x; 1.2620x over previous
import functools

import jax
import jax.numpy as jnp
from jax import lax
from jax.experimental import pallas as pl
from jax.experimental.pallas import tpu as pltpu
from jax.experimental.pallas import tpu_sc as plsc

EPS = 1e-09
N_TASKS = 1000
N_SKILLS = 64
N_SPLITS = 8
BS = 16384
D = N_SKILLS * N_SPLITS

NUM_CORES = 2
NUM_SUBCORES = 16
NUM_WORKERS = NUM_CORES * NUM_SUBCORES
B_PER_W = BS // NUM_WORKERS
CHUNK = 64
N_CHUNKS = B_PER_W // CHUNK


def _normalize_body(w_ref, out_ref):
    s = jax.nn.sigmoid(w_ref[...])
    denom = jnp.sum(s, axis=1, keepdims=True) + EPS
    mw = s / denom
    out_ref[...] = jnp.concatenate([mw, mw], axis=1)


def _normalize_table(w2):
    return pl.pallas_call(
        _normalize_body,
        out_shape=jax.ShapeDtypeStruct((w2.shape[0], 2 * N_SKILLS), w2.dtype),
    )(w2)


_mesh = plsc.VectorSubcoreMesh(core_axis_name="c", subcore_axis_name="s")


@functools.partial(
    pl.kernel,
    mesh=_mesh,
    out_type=jax.ShapeDtypeStruct((BS, N_SPLITS, 2 * N_SKILLS), jnp.float32),
    scratch_types=[
        pltpu.VMEM((CHUNK,), jnp.int32),
        pltpu.VMEM((CHUNK, N_SPLITS, 2 * N_SKILLS), jnp.float32),
        pltpu.SemaphoreType.DMA,
    ],
)
def _sc_gather(idx_hbm, table_hbm, out_hbm, idx_v, rows_v, sem):
    wid = lax.axis_index("s") * NUM_CORES + lax.axis_index("c")
    base = wid * B_PER_W
    for c in range(N_CHUNKS):
        off = base + c * CHUNK
        pltpu.sync_copy(idx_hbm.at[pl.ds(off, CHUNK)], idx_v)
        pltpu.async_copy(table_hbm.at[idx_v], rows_v, sem).wait()
        pltpu.sync_copy(rows_v, out_hbm.at[pl.ds(off, CHUNK)])


def kernel(routing_info, weights):
    w2 = weights.reshape(N_TASKS * N_SPLITS, N_SKILLS)
    table = _normalize_table(w2).reshape(N_TASKS, N_SPLITS, 2 * N_SKILLS)
    idx = routing_info.reshape(BS).astype(jnp.int32)
    return _sc_gather(idx, table)[:, :, :N_SKILLS]


# speedup vs baseline: 1.5880x; 1.2583x over previous
import functools

import jax
import jax.numpy as jnp
from jax import lax
from jax.experimental import pallas as pl
from jax.experimental.pallas import tpu as pltpu
from jax.experimental.pallas import tpu_sc as plsc

EPS = 1e-09
N_TASKS = 1000
N_SKILLS = 64
N_SPLITS = 8
BS = 16384
D = N_SKILLS * N_SPLITS

NUM_CORES = 2
NUM_SUBCORES = 16
NUM_WORKERS = NUM_CORES * NUM_SUBCORES
B_PER_W = BS // NUM_WORKERS
CHUNK = 128
N_CHUNKS = B_PER_W // CHUNK

T_PAD = 1008
T_PER_SUB = 64
LANES = 16

_mesh = plsc.VectorSubcoreMesh(core_axis_name="c", subcore_axis_name="s")


@functools.partial(
    pl.kernel,
    mesh=_mesh,
    out_type=[
        jax.ShapeDtypeStruct((BS, D), jnp.float32),
        jax.ShapeDtypeStruct((T_PAD, D), jnp.float32),
    ],
    scratch_types=[
        pltpu.VMEM((T_PER_SUB, D), jnp.float32),
        pltpu.VMEM((CHUNK,), jnp.int32),
        pltpu.VMEM((CHUNK, D), jnp.float32),
        pltpu.SemaphoreType.DMA,
    ],
    compiler_params=pltpu.CompilerParams(needs_layout_passes=False),
)
def _sc_run(idx_hbm, wpad_hbm, out_hbm, table_hbm, wv, idx_v, rows_v, sem):
    cid = lax.axis_index("c")
    sid = lax.axis_index("s")

    trow = pl.multiple_of(
        jnp.minimum(sid * T_PER_SUB, T_PAD - T_PER_SUB), 8
    )
    pltpu.sync_copy(wpad_hbm.at[pl.ds(trow, T_PER_SUB)], wv)

    def norm_row(r, _):
        for g in range(N_SPLITS):
            base = g * N_SKILLS
            xs = [wv[r, pl.ds(base + j * LANES, LANES)] for j in range(4)]
            ss = [1.0 / (1.0 + jnp.exp(-x)) for x in xs]
            tot = jnp.sum(ss[0] + ss[1] + ss[2] + ss[3])
            inv = 1.0 / (jnp.full((LANES,), tot, jnp.float32) + EPS)
            for j in range(4):
                wv[r, pl.ds(base + j * LANES, LANES)] = ss[j] * inv
        return _

    lax.fori_loop(0, T_PER_SUB, norm_row, None)

    pltpu.sync_copy(wv, table_hbm.at[pl.ds(trow, T_PER_SUB)])
    plsc.subcore_barrier()

    wid = sid * NUM_CORES + cid
    bbase = wid * B_PER_W
    for c in range(N_CHUNKS):
        off = bbase + c * CHUNK
        pltpu.sync_copy(idx_hbm.at[pl.ds(off, CHUNK)], idx_v)
        pltpu.async_copy(table_hbm.at[idx_v], rows_v, sem).wait()
        pltpu.sync_copy(rows_v, out_hbm.at[pl.ds(off, CHUNK)])


def kernel(routing_info, weights):
    wpad = jnp.concatenate(
        [weights, jnp.zeros((T_PAD - N_TASKS, D), jnp.float32)], axis=0
    )
    idx = routing_info.reshape(BS).astype(jnp.int32)
    out, _ = _sc_run(idx, wpad)
    return out.reshape(BS, N_SPLITS, N_SKILLS)
